# baseline (speedup 1.0000x reference)
import jax
import jax.numpy as jnp

def kernel(data, d, edge_index, W0, U0, G0, W1, U1, G1, W2, U2, G2,
           fW1, fb1, fW2, fb2, fW3, fb3, fW4, fb4):
    parts = [data[0, 0], d[0, 0], jnp.float32(edge_index[0, 0]), W0[0, 0],
             U0[0, 0], G0[0, 0], W1[0, 0], U1[0, 0], G1[0, 0], W2[0, 0],
             U2[0, 0], G2[0, 0], fW1[0, 0], fb1[0], fW2[0, 0], fb2[0],
             fW3[0, 0], fb3[0], fW4[0, 0], fb4[0]]
    s = parts[0]
    for p in parts[1:]:
        s = s + p
    return s


# speedup vs baseline: 1.1356x; 1.1356x over previous
import jax
import jax.numpy as jnp
from jax.experimental import pallas as pl
from jax.experimental.pallas import tpu as pltpu

N_NODES = 50
N_EDGES = 800
EMB = 5
F32 = jnp.float32


def _lrelu(x):
    return jnp.where(x >= 0, x, 0.01 * x)


def _sigmoid(x):
    return 1.0 / (1.0 + jnp.exp(-x))


def _dot(a, b):
    return jax.lax.dot_general(a, b, (((1,), (0,)), ((), ())),
                               preferred_element_type=F32)


def _fused(ei_ref, fW1_ref, g128_ref, g64_ref, g5_ref, fW4_ref, fb4_ref,
           d_ref, data_ref, out_ref):
    src = ei_ref[0:1, :]
    dst = ei_ref[1:2, :]
    rows = jax.lax.broadcasted_iota(jnp.int32, (N_NODES, N_EDGES), 0)
    m_dst = (rows == dst).astype(F32)
    m_src = (rows == src).astype(F32)
    A = jax.lax.dot_general(m_dst, m_src, (((1,), (1,)), ((), ())),
                            preferred_element_type=F32)

    x0 = data_ref[...]
    W0 = g5_ref[0:1, :]
    U0 = g5_ref[1:2, :]
    G0 = g5_ref[2:3, :]
    ax0 = _dot(A, x0)
    t0 = ax0 * W0
    h = _lrelu(x0 * U0 + t0)
    g = _sigmoid(x0 * G0 + t0)
    x = jnp.concatenate([h, g * h], axis=1)

    for base in (3, 33):
        W = g5_ref[base:base + 10, :]
        U = g5_ref[base + 10:base + 20, :]
        G = g5_ref[base + 20:base + 30, :]
        ax = _dot(A, x)
        t = _dot(ax, W)
        h = _lrelu(_dot(x, U) + t)
        g = _sigmoid(_dot(x, G) + t)
        x = jnp.concatenate([h, g * h], axis=1)

    dmat = d_ref[...]
    pieces = ([x[i:i + 1, :] for i in range(N_NODES)]
              + [dmat[i:i + 1, :] for i in range(N_NODES)])
    full = jnp.concatenate(pieces, axis=1)

    fW2 = g128_ref[0:128, :]
    fb1 = g128_ref[128:129, :]
    fb2 = g128_ref[129:130, :]
    fW3 = g64_ref[0:128, :]
    fb3 = g64_ref[128:129, :]
    h1 = _lrelu(_dot(full, fW1_ref[...]) + fb1)
    h2 = _lrelu(_dot(h1, fW2) + fb2)
    h3 = _lrelu(_dot(h2, fW3) + fb3)
    y = _sigmoid(_dot(h3, fW4_ref[...]) + fb4_ref[...])
    out_ref[...] = 0.5 * (y[0, 0] + y[0, 1])


def kernel(data, d, edge_index, W0, U0, G0, W1, U1, G1, W2, U2, G2,
           fW1, fb1, fW2, fb2, fW3, fb3, fW4, fb4):
    g128 = jnp.concatenate(
        [fW2, fb1.reshape(1, -1), fb2.reshape(1, -1)], axis=0)
    g64 = jnp.concatenate([fW3, fb3.reshape(1, -1)], axis=0)
    g5 = jnp.concatenate(
        [W0, U0, G0, W1, U1, G1, W2, U2, G2], axis=0)
    out = pl.pallas_call(
        _fused,
        out_shape=jax.ShapeDtypeStruct((), F32),
        out_specs=pl.BlockSpec(memory_space=pltpu.SMEM),
    )(edge_index, fW1, g128, g64, g5, fW4, fb4.reshape(1, -1), d, data)
    return out


# speedup vs baseline: 1.9775x; 1.7414x over previous
import jax
import jax.numpy as jnp
from jax.experimental import pallas as pl
from jax.experimental.pallas import tpu as pltpu

N_NODES = 50
N_EDGES = 800
EMB = 5
F32 = jnp.float32


def _lrelu(x):
    return jnp.where(x >= 0, x, 0.01 * x)


def _sigmoid(x):
    return 1.0 / (1.0 + jnp.exp(-x))


def _dot(a, b):
    return jax.lax.dot_general(a, b, (((1,), (0,)), ((), ())),
                               preferred_element_type=F32)


def _fused(ei_ref, data_ref, dflat_ref,
           W0_ref, U0_ref, G0_ref, W1_ref, U1_ref, G1_ref, W2_ref, U2_ref,
           G2_ref, fW1p_ref, fb1_ref, fW2_ref, fb2_ref, fW3_ref, fb3_ref,
           fW4_ref, fb4_ref, out_ref):
    src = ei_ref[0:1, :]
    dst = ei_ref[1:2, :]
    rows = jax.lax.broadcasted_iota(jnp.int32, (N_NODES, N_EDGES), 0)
    m_dst = (rows == dst).astype(F32)
    m_src = (rows == src).astype(F32)
    A = jax.lax.dot_general(m_dst, m_src, (((1,), (1,)), ((), ())),
                            preferred_element_type=F32)

    x0 = data_ref[...]
    ax0 = _dot(A, x0)
    t0 = ax0 * W0_ref[...]
    h = _lrelu(x0 * U0_ref[...] + t0)
    g = _sigmoid(x0 * G0_ref[...] + t0)
    x = jnp.concatenate([h, g * h], axis=1)

    for W_ref, U_ref, G_ref in ((W1_ref, U1_ref, G1_ref),
                                (W2_ref, U2_ref, G2_ref)):
        ax = _dot(A, x)
        t = _dot(ax, W_ref[...])
        h = _lrelu(_dot(x, U_ref[...]) + t)
        g = _sigmoid(_dot(x, G_ref[...]) + t)
        x = jnp.concatenate([h, g * h], axis=1)

    xt = jnp.transpose(x)
    vecx = jnp.concatenate([xt[j:j + 1, :] for j in range(2 * EMB)], axis=1)
    full = jnp.concatenate([vecx, dflat_ref[...]], axis=1)

    h1 = _lrelu(_dot(full, fW1p_ref[...]) + fb1_ref[...])
    h2 = _lrelu(_dot(h1, fW2_ref[...]) + fb2_ref[...])
    h3 = _lrelu(_dot(h2, fW3_ref[...]) + fb3_ref[...])
    y = _sigmoid(_dot(h3, fW4_ref[...]) + fb4_ref[...])
    out_ref[...] = 0.5 * (y[0, 0] + y[0, 1])


def kernel(data, d, edge_index, W0, U0, G0, W1, U1, G1, W2, U2, G2,
           fW1, fb1, fW2, fb2, fW3, fb3, fW4, fb4):
    dflat = d.reshape(1, -1)
    fW1x = fW1[:N_NODES * 2 * EMB].reshape(N_NODES, 2 * EMB, -1)
    fW1p = jnp.concatenate(
        [fW1x.transpose(1, 0, 2).reshape(N_NODES * 2 * EMB, -1),
         fW1[N_NODES * 2 * EMB:]], axis=0)
    out = pl.pallas_call(
        _fused,
        out_shape=jax.ShapeDtypeStruct((), F32),
        out_specs=pl.BlockSpec(memory_space=pltpu.SMEM),
    )(edge_index, data, dflat, W0, U0, G0, W1, U1, G1, W2, U2, G2,
      fW1p, fb1.reshape(1, -1), fW2, fb2.reshape(1, -1),
      fW3, fb3.reshape(1, -1), fW4, fb4.reshape(1, -1))
    return out
